# baseline (speedup 1.0000x reference)
import functools

import jax
import jax.numpy as jnp
from jax import lax
from jax.experimental import pallas as pl
from jax.experimental.pallas import tpu as pltpu
from jax.experimental.pallas import tpu_sc as plsc

N = 100000
NPAD = 100352
TPR = NPAD // 16
QC = TPR // 8
E = 1600000
EROWS = 12544
EPAD = EROWS * 128
RPT = EROWS // 16
GB = 4
NGRP = RPT // GB
RPW = EROWS // 32
GA = 8
NPARTS = 4
ZROWS = NPARTS * NPAD

_mesh = plsc.VectorSubcoreMesh(core_axis_name="c", subcore_axis_name="s")
_sc_params = pltpu.CompilerParams(use_tc_tiling_on_sc=False,
                                  needs_layout_passes=False)


@functools.partial(
    pl.kernel,
    out_type=jax.ShapeDtypeStruct((2, NPAD), jnp.float32),
    mesh=_mesh,
    compiler_params=_sc_params,
    scratch_types=[
        pltpu.MemorySpace.VMEM_SHARED((NPAD, 16), jnp.float32),
        pltpu.VMEM((QC, 16), jnp.float32),
        pltpu.VMEM((TPR,), jnp.float32),
        pltpu.VMEM((GA, 128), jnp.int32),
    ],
)
def _deg_kernel(dst_hbm, out_hbm, d_sh, stage_v, deg_v, dst_v):
    c = lax.axis_index("c")
    t = lax.axis_index("s")
    ones = jnp.ones((16,), jnp.float32)

    def fill(i, _):
        stage_v[i] = ones
        return _

    lax.fori_loop(0, QC, fill, None)
    for q in range(8):
        pltpu.sync_copy(stage_v, d_sh.at[pl.ds(t * TPR + q * QC, QC)])
    plsc.subcore_barrier()

    base = c * (EROWS // 2) + t * RPW

    def body(g, _):
        row0 = base + g * GA
        pltpu.sync_copy(dst_hbm.at[pl.ds(row0, GA)], dst_v)
        for j in range(GA):
            pltpu.sync_copy(stage_v.at[pl.ds(0, 128)], d_sh.at[dst_v.at[j]],
                            add=True)
        return _

    lax.fori_loop(0, RPW // GA, body, None)
    plsc.subcore_barrier()

    rows16 = lax.iota(jnp.int32, 16)
    zeros16 = jnp.zeros((16,), jnp.int32)

    def compact(q):
        pltpu.sync_copy(d_sh.at[pl.ds(t * TPR + q * QC, QC)], stage_v)

        def inner(w, _):
            vals = plsc.load_gather(stage_v, [w * 16 + rows16, zeros16])
            deg_v[pl.ds(q * QC + w * 16, 16)] = vals
            return _

        lax.fori_loop(0, QC // 16, inner, None)

    for q in range(8):
        compact(q)
    pltpu.sync_copy(deg_v, out_hbm.at[c, pl.ds(t * TPR, TPR)])


@functools.partial(
    pl.kernel,
    out_type=jax.ShapeDtypeStruct((ZROWS, 16), jnp.float32),
    mesh=_mesh,
    compiler_params=_sc_params,
    scratch_types=[
        pltpu.MemorySpace.VMEM_SHARED((NPAD, 16), jnp.float32),
        pltpu.VMEM((2, GB, 128), jnp.int32),
        pltpu.VMEM((2, GB, 128), jnp.int32),
        pltpu.VMEM((2, GB, 128, 16), jnp.float32),
        pltpu.SemaphoreType.DMA,
        pltpu.SemaphoreType.DMA,
        pltpu.SemaphoreType.DMA,
        pltpu.SemaphoreType.DMA,
    ],
)
def _segsum_kernel(zflat_hbm, srco_hbm, dst_hbm, out_hbm,
                   s_sh, src_v, dst_v, rows_v, gsem0, gsem1, ssem0, ssem1):
    c = lax.axis_index("c")
    t = lax.axis_index("s")
    gsem = (gsem0, gsem1)
    ssem = (ssem0, ssem1)

    def drain(sem, buf):
        for j in range(GB):
            pltpu.make_async_copy(zflat_hbm.at[pl.ds(0, 128)],
                                  rows_v.at[buf, j], sem).wait()

    for p in range(2):
        part = c * 2 + p
        zoff = part * NPAD
        pltpu.sync_copy(zflat_hbm.at[pl.ds(zoff + t * TPR, TPR)],
                        s_sh.at[pl.ds(t * TPR, TPR)])
        plsc.subcore_barrier()

        def load_and_fire(sb, buf):
            row0 = t * RPT + sb * GB
            pltpu.sync_copy(srco_hbm.at[part, pl.ds(row0, GB)], src_v.at[buf])
            pltpu.sync_copy(dst_hbm.at[pl.ds(row0, GB)], dst_v.at[buf])
            for j in range(GB):
                pltpu.async_copy(zflat_hbm.at[src_v.at[buf, j]],
                                 rows_v.at[buf, j], gsem[buf])

        load_and_fire(0, 0)

        def body(g, _):
            for b in range(2):
                sb = g + b
                drain(gsem[b], b)
                nxt = sb + 1

                @pl.when((nxt < NGRP) & (sb >= 1))
                def _():
                    drain(ssem[1 - b], 1 - b)

                @pl.when(nxt < NGRP)
                def _():
                    load_and_fire(nxt, 1 - b)

                for j in range(GB):
                    pltpu.async_copy(rows_v.at[b, j], s_sh.at[dst_v.at[b, j]],
                                     ssem[b], add=True)
            return _

        lax.fori_loop(0, NGRP // 2, lambda i, u: body(i * 2, u), None)
        drain(ssem[0], 0)
        drain(ssem[1], 1)
        plsc.subcore_barrier()
        pltpu.sync_copy(s_sh.at[pl.ds(t * TPR, TPR)],
                        out_hbm.at[pl.ds(zoff + t * TPR, TPR)])
        plsc.subcore_barrier()


def _unpack_lanes(blk, w):
    return jnp.concatenate(
        [blk[:, u * w:(u + 1) * w] for u in range(8)], axis=0)


def _scale_body(x_ref, dpk_ref, out_ref):
    dinv = _unpack_lanes(dpk_ref[...], 16)[:, 0:1]
    z = x_ref[...] * dinv
    for p in range(NPARTS):
        zp = z[:, p * 16:(p + 1) * 16]
        out_ref[p] = jnp.concatenate(
            [zp[u * (TPR // 8):(u + 1) * (TPR // 8), :] for u in range(8)],
            axis=1)


def _scale_split(x_pad, dinvpk):
    return pl.pallas_call(
        _scale_body,
        grid=(NPAD // TPR,),
        in_specs=[
            pl.BlockSpec((TPR, 64), lambda i: (i, 0)),
            pl.BlockSpec((TPR // 8, 128), lambda i: (i, 0)),
        ],
        out_specs=pl.BlockSpec((NPARTS, TPR // 8, 128), lambda i: (0, i, 0)),
        out_shape=jax.ShapeDtypeStruct((NPARTS, NPAD // 8, 128), jnp.float32),
    )(x_pad, dinvpk)


def _head_body(st_ref, dpk_ref, wc_ref, bc_ref, wl_ref, bl_ref,
               out_ref, acc_ref):
    i = pl.program_id(0)

    g = jnp.dot(_unpack_lanes(st_ref[0], 16), wc_ref[0],
                preferred_element_type=jnp.float32)
    for p in range(1, NPARTS):
        g += jnp.dot(_unpack_lanes(st_ref[p], 16), wc_ref[p],
                     preferred_element_type=jnp.float32)
    dinv = _unpack_lanes(dpk_ref[...], 16)[:, 0:1]
    h = jnp.maximum(g * dinv + bc_ref[...], 0.0)
    rows = lax.broadcasted_iota(jnp.int32, (TPR, 1), 0) + i * TPR
    h = jnp.where(rows < N, h, 0.0)
    partial = jnp.sum(h.reshape(TPR // 8, 8, 128), axis=0)

    @pl.when(i == 0)
    def _():
        acc_ref[...] = partial

    @pl.when(i > 0)
    def _():
        acc_ref[...] += partial

    @pl.when(i == (NPAD // TPR) - 1)
    def _():
        emb = jnp.sum(acc_ref[...], axis=0, keepdims=True) * (1.0 / N)
        out_ref[...] = jnp.tanh(
            jnp.dot(emb, wl_ref[...], preferred_element_type=jnp.float32)
            + bl_ref[...])


def _head(stpk, dinvpk, wc4, bc2d, wl, bl2d):
    return pl.pallas_call(
        _head_body,
        grid=(NPAD // TPR,),
        in_specs=[
            pl.BlockSpec((NPARTS, TPR // 8, 128), lambda i: (0, i, 0)),
            pl.BlockSpec((TPR // 8, 128), lambda i: (i, 0)),
            pl.BlockSpec((NPARTS, 16, 128), lambda i: (0, 0, 0)),
            pl.BlockSpec((1, 128), lambda i: (0, 0)),
            pl.BlockSpec((128, 128), lambda i: (0, 0)),
            pl.BlockSpec((1, 128), lambda i: (0, 0)),
        ],
        out_specs=pl.BlockSpec((1, 128), lambda i: (0, 0)),
        out_shape=jax.ShapeDtypeStruct((1, 128), jnp.float32),
        scratch_shapes=[pltpu.VMEM((8, 128), jnp.float32)],
    )(stpk, dinvpk, wc4, bc2d, wl, bl2d)


def kernel(edge_index, W_conv, b_conv, W_lin, b_lin):
    in_feat = W_conv.shape[0]
    src = edge_index[0].astype(jnp.int32)
    dst = edge_index[1].astype(jnp.int32)
    npad_e = EPAD - E
    pad_src = N + (jnp.arange(npad_e, dtype=jnp.int32) % 8)
    pad_dst = (N + 8) + (jnp.arange(npad_e, dtype=jnp.int32) % (NPAD - N - 8))
    def perm(n):
        blk, r = n // TPR, n % TPR
        return blk * TPR + 8 * (r % (TPR // 8)) + r // (TPR // 8)

    dst_plain = jnp.concatenate([dst, pad_dst])
    srcp = perm(jnp.concatenate([src, pad_src]))
    srco = (srcp[None, :]
            + (jnp.arange(NPARTS, dtype=jnp.int32) * NPAD)[:, None]
            ).reshape(NPARTS, EROWS, 128)
    dst2d = perm(dst_plain).reshape(EROWS, 128)

    degc = _deg_kernel(dst_plain.reshape(EROWS, 128))
    deg = degc[0] + degc[1] - 1.0
    dinv1d = lax.rsqrt(deg)
    dinvpk = jnp.repeat(
        jnp.swapaxes(dinv1d.reshape(16, 8, TPR // 8), 1, 2).reshape(-1), 16
    ).reshape(NPAD // 8, 128)

    x = jax.random.normal(jax.random.key(42), (N, in_feat), dtype=jnp.float32)
    x_pad = jnp.zeros((NPAD, 64), jnp.float32).at[:N, :in_feat].set(x)

    zpk = _scale_split(x_pad, dinvpk)
    st = _segsum_kernel(zpk.reshape(ZROWS, 16), srco, dst2d)
    stpk = st.reshape(NPARTS, NPAD // 8, 128)

    wc4 = jnp.zeros((64, 128), jnp.float32).at[:in_feat].set(W_conv)
    wc4 = wc4.reshape(NPARTS, 16, 128)
    out = _head(stpk, dinvpk, wc4, b_conv[None, :], W_lin, b_lin[None, :])
    return out


# speedup vs baseline: 1.1527x; 1.1527x over previous
import functools

import jax
import jax.numpy as jnp
from jax import lax
from jax.experimental import pallas as pl
from jax.experimental.pallas import tpu as pltpu
from jax.experimental.pallas import tpu_sc as plsc

N = 100000
NPAD = 100352
TPR = NPAD // 16
QC = TPR // 8
E = 1600000
EROWS = 12544
EPAD = EROWS * 128
RPT = EROWS // 16
GB = 8
NGRP = RPT // GB
RPW = EROWS // 32
GA = 8
NPARTS = 4
ZROWS = NPARTS * NPAD

_mesh = plsc.VectorSubcoreMesh(core_axis_name="c", subcore_axis_name="s")
_sc_params = pltpu.CompilerParams(use_tc_tiling_on_sc=False,
                                  needs_layout_passes=False)


@functools.partial(
    pl.kernel,
    out_type=jax.ShapeDtypeStruct((2, NPAD), jnp.float32),
    mesh=_mesh,
    compiler_params=_sc_params,
    scratch_types=[
        pltpu.MemorySpace.VMEM_SHARED((NPAD, 16), jnp.float32),
        pltpu.VMEM((QC, 16), jnp.float32),
        pltpu.VMEM((TPR,), jnp.float32),
        pltpu.VMEM((GA, 128), jnp.int32),
    ],
)
def _deg_kernel(dst_hbm, out_hbm, d_sh, stage_v, deg_v, dst_v):
    c = lax.axis_index("c")
    t = lax.axis_index("s")
    ones = jnp.ones((16,), jnp.float32)

    def fill(i, _):
        stage_v[i] = ones
        return _

    lax.fori_loop(0, QC, fill, None)
    for q in range(8):
        pltpu.sync_copy(stage_v, d_sh.at[pl.ds(t * TPR + q * QC, QC)])
    plsc.subcore_barrier()

    base = c * (EROWS // 2) + t * RPW

    def body(g, _):
        row0 = base + g * GA
        pltpu.sync_copy(dst_hbm.at[pl.ds(row0, GA)], dst_v)
        for j in range(GA):
            pltpu.sync_copy(stage_v.at[pl.ds(0, 128)], d_sh.at[dst_v.at[j]],
                            add=True)
        return _

    lax.fori_loop(0, RPW // GA, body, None)
    plsc.subcore_barrier()

    rows16 = lax.iota(jnp.int32, 16)
    zeros16 = jnp.zeros((16,), jnp.int32)

    def compact(q):
        pltpu.sync_copy(d_sh.at[pl.ds(t * TPR + q * QC, QC)], stage_v)

        def inner(w, _):
            vals = plsc.load_gather(stage_v, [w * 16 + rows16, zeros16])
            deg_v[pl.ds(q * QC + w * 16, 16)] = vals
            return _

        lax.fori_loop(0, QC // 16, inner, None)

    for q in range(8):
        compact(q)
    pltpu.sync_copy(deg_v, out_hbm.at[c, pl.ds(t * TPR, TPR)])


@functools.partial(
    pl.kernel,
    out_type=jax.ShapeDtypeStruct((ZROWS, 16), jnp.float32),
    mesh=_mesh,
    compiler_params=_sc_params,
    scratch_types=[
        pltpu.MemorySpace.VMEM_SHARED((NPAD, 16), jnp.float32),
        pltpu.VMEM((GB, 128), jnp.int32),
        pltpu.VMEM((GB, 128), jnp.int32),
        pltpu.VMEM((GB, 128, 16), jnp.float32),
        pltpu.SemaphoreType.DMA,
    ],
)
def _segsum_kernel(zflat_hbm, src_hbm, dst_hbm, out_hbm,
                   s_sh, src_v, dst_v, rows_v, sem):
    c = lax.axis_index("c")
    t = lax.axis_index("s")
    for p in range(2):
        part = c * 2 + p
        zoff = part * NPAD
        pltpu.sync_copy(zflat_hbm.at[pl.ds(zoff + t * TPR, TPR)],
                        s_sh.at[pl.ds(t * TPR, TPR)])
        plsc.subcore_barrier()

        def body(g, _):
            row0 = t * RPT + g * GB
            pltpu.sync_copy(src_hbm.at[pl.ds(row0, GB)], src_v)
            pltpu.sync_copy(dst_hbm.at[pl.ds(row0, GB)], dst_v)
            for j in range(GB):
                for k in range(8):
                    sl = (j, pl.ds(k * 16, 16))
                    src_v[sl] = src_v[sl] + zoff
            handles = [
                pltpu.async_copy(zflat_hbm.at[src_v.at[j]], rows_v.at[j], sem)
                for j in range(GB)
            ]
            for h in handles:
                h.wait()
            for j in range(GB):
                pltpu.sync_copy(rows_v.at[j], s_sh.at[dst_v.at[j]], add=True)
            return _

        lax.fori_loop(0, NGRP, body, None)
        plsc.subcore_barrier()
        pltpu.sync_copy(s_sh.at[pl.ds(t * TPR, TPR)],
                        out_hbm.at[pl.ds(zoff + t * TPR, TPR)])
        plsc.subcore_barrier()


def _unpack_lanes(blk, w):
    return jnp.concatenate(
        [blk[:, u * w:(u + 1) * w] for u in range(8)], axis=0)


def _scale_body(x_ref, dpk_ref, out_ref):
    dinv = _unpack_lanes(dpk_ref[...], 16)[:, 0:1]
    z = x_ref[...] * dinv
    for p in range(NPARTS):
        zp = z[:, p * 16:(p + 1) * 16]
        out_ref[p] = jnp.concatenate(
            [zp[u * (TPR // 8):(u + 1) * (TPR // 8), :] for u in range(8)],
            axis=1)


def _scale_split(x_pad, dinvpk):
    return pl.pallas_call(
        _scale_body,
        grid=(NPAD // TPR,),
        in_specs=[
            pl.BlockSpec((TPR, 64), lambda i: (i, 0)),
            pl.BlockSpec((TPR // 8, 128), lambda i: (i, 0)),
        ],
        out_specs=pl.BlockSpec((NPARTS, TPR // 8, 128), lambda i: (0, i, 0)),
        out_shape=jax.ShapeDtypeStruct((NPARTS, NPAD // 8, 128), jnp.float32),
    )(x_pad, dinvpk)


def _head_body(st_ref, dpk_ref, wc_ref, bc_ref, wl_ref, bl_ref,
               out_ref, acc_ref):
    i = pl.program_id(0)

    g = jnp.dot(_unpack_lanes(st_ref[0], 16), wc_ref[0],
                preferred_element_type=jnp.float32)
    for p in range(1, NPARTS):
        g += jnp.dot(_unpack_lanes(st_ref[p], 16), wc_ref[p],
                     preferred_element_type=jnp.float32)
    dinv = _unpack_lanes(dpk_ref[...], 16)[:, 0:1]
    h = jnp.maximum(g * dinv + bc_ref[...], 0.0)
    rows = lax.broadcasted_iota(jnp.int32, (TPR, 1), 0) + i * TPR
    h = jnp.where(rows < N, h, 0.0)
    partial = jnp.sum(h.reshape(TPR // 8, 8, 128), axis=0)

    @pl.when(i == 0)
    def _():
        acc_ref[...] = partial

    @pl.when(i > 0)
    def _():
        acc_ref[...] += partial

    @pl.when(i == (NPAD // TPR) - 1)
    def _():
        emb = jnp.sum(acc_ref[...], axis=0, keepdims=True) * (1.0 / N)
        out_ref[...] = jnp.tanh(
            jnp.dot(emb, wl_ref[...], preferred_element_type=jnp.float32)
            + bl_ref[...])


def _head(stpk, dinvpk, wc4, bc2d, wl, bl2d):
    return pl.pallas_call(
        _head_body,
        grid=(NPAD // TPR,),
        in_specs=[
            pl.BlockSpec((NPARTS, TPR // 8, 128), lambda i: (0, i, 0)),
            pl.BlockSpec((TPR // 8, 128), lambda i: (i, 0)),
            pl.BlockSpec((NPARTS, 16, 128), lambda i: (0, 0, 0)),
            pl.BlockSpec((1, 128), lambda i: (0, 0)),
            pl.BlockSpec((128, 128), lambda i: (0, 0)),
            pl.BlockSpec((1, 128), lambda i: (0, 0)),
        ],
        out_specs=pl.BlockSpec((1, 128), lambda i: (0, 0)),
        out_shape=jax.ShapeDtypeStruct((1, 128), jnp.float32),
        scratch_shapes=[pltpu.VMEM((8, 128), jnp.float32)],
    )(stpk, dinvpk, wc4, bc2d, wl, bl2d)


def kernel(edge_index, W_conv, b_conv, W_lin, b_lin):
    in_feat = W_conv.shape[0]
    src = edge_index[0].astype(jnp.int32)
    dst = edge_index[1].astype(jnp.int32)
    npad_e = EPAD - E
    pad_src = N + (jnp.arange(npad_e, dtype=jnp.int32) % 8)
    pad_dst = (N + 8) + (jnp.arange(npad_e, dtype=jnp.int32) % (NPAD - N - 8))
    def perm(n):
        blk, r = n // TPR, n % TPR
        return blk * TPR + 8 * (r % (TPR // 8)) + r // (TPR // 8)

    dst_plain = jnp.concatenate([dst, pad_dst])
    srcp = perm(jnp.concatenate([src, pad_src]))
    src2d = srcp.reshape(EROWS, 128)
    dst2d = perm(dst_plain).reshape(EROWS, 128)

    degc = _deg_kernel(dst_plain.reshape(EROWS, 128))
    deg = degc[0] + degc[1] - 1.0
    dinv1d = lax.rsqrt(deg)
    dinvpk = jnp.repeat(
        jnp.swapaxes(dinv1d.reshape(16, 8, TPR // 8), 1, 2).reshape(-1), 16
    ).reshape(NPAD // 8, 128)

    x = jax.random.normal(jax.random.key(42), (N, in_feat), dtype=jnp.float32)
    x_pad = jnp.zeros((NPAD, 64), jnp.float32).at[:N, :in_feat].set(x)

    zpk = _scale_split(x_pad, dinvpk)
    st = _segsum_kernel(zpk.reshape(ZROWS, 16), src2d, dst2d)
    stpk = st.reshape(NPARTS, NPAD // 8, 128)

    wc4 = jnp.zeros((64, 128), jnp.float32).at[:in_feat].set(W_conv)
    wc4 = wc4.reshape(NPARTS, 16, 128)
    out = _head(stpk, dinvpk, wc4, b_conv[None, :], W_lin, b_lin[None, :])
    return out


# speedup vs baseline: 1.2320x; 1.0688x over previous
import functools

import jax
import jax.numpy as jnp
from jax import lax
from jax.experimental import pallas as pl
from jax.experimental.pallas import tpu as pltpu
from jax.experimental.pallas import tpu_sc as plsc

N = 100000
NPAD = 100352
TPR = NPAD // 16
QC = TPR // 8
E = 1600000
EROWS = 12544
EPAD = EROWS * 128
RPT = EROWS // 16
GB = 8
NGRP = RPT // GB
RPW = EROWS // 32
GA = 8
NPARTS = 4
ZROWS = NPARTS * NPAD

_mesh = plsc.VectorSubcoreMesh(core_axis_name="c", subcore_axis_name="s")
_sc_params = pltpu.CompilerParams(use_tc_tiling_on_sc=False,
                                  needs_layout_passes=False)


@functools.partial(
    pl.kernel,
    out_type=jax.ShapeDtypeStruct((2, NPAD), jnp.float32),
    mesh=_mesh,
    compiler_params=_sc_params,
    scratch_types=[
        pltpu.MemorySpace.VMEM_SHARED((NPAD, 16), jnp.float32),
        pltpu.VMEM((QC, 16), jnp.float32),
        pltpu.VMEM((TPR,), jnp.float32),
        pltpu.VMEM((GA, 128), jnp.int32),
    ],
)
def _deg_kernel(dst_hbm, out_hbm, d_sh, stage_v, deg_v, dst_v):
    c = lax.axis_index("c")
    t = lax.axis_index("s")
    ones = jnp.ones((16,), jnp.float32)

    def fill(i, _):
        stage_v[i] = ones
        return _

    lax.fori_loop(0, QC, fill, None)
    for q in range(8):
        pltpu.sync_copy(stage_v, d_sh.at[pl.ds(t * TPR + q * QC, QC)])
    plsc.subcore_barrier()

    base = c * (EROWS // 2) + t * RPW

    def body(g, _):
        row0 = base + g * GA
        pltpu.sync_copy(dst_hbm.at[pl.ds(row0, GA)], dst_v)
        for j in range(GA):
            pltpu.sync_copy(stage_v.at[pl.ds(0, 128)], d_sh.at[dst_v.at[j]],
                            add=True)
        return _

    lax.fori_loop(0, RPW // GA, body, None)
    plsc.subcore_barrier()

    rows16 = lax.iota(jnp.int32, 16)
    zeros16 = jnp.zeros((16,), jnp.int32)

    def compact(q):
        pltpu.sync_copy(d_sh.at[pl.ds(t * TPR + q * QC, QC)], stage_v)

        def inner(w, _):
            vals = plsc.load_gather(stage_v, [w * 16 + rows16, zeros16])
            deg_v[pl.ds(q * QC + w * 16, 16)] = vals
            return _

        lax.fori_loop(0, QC // 16, inner, None)

    for q in range(8):
        compact(q)
    pltpu.sync_copy(deg_v, out_hbm.at[c, pl.ds(t * TPR, TPR)])


@functools.partial(
    pl.kernel,
    out_type=jax.ShapeDtypeStruct((ZROWS, 16), jnp.float32),
    mesh=_mesh,
    compiler_params=_sc_params,
    scratch_types=[
        pltpu.MemorySpace.VMEM_SHARED((NPAD, 16), jnp.float32),
        pltpu.VMEM((GB, 128), jnp.int32),
        pltpu.VMEM((GB, 128), jnp.int32),
        pltpu.VMEM((GB, 128, 16), jnp.float32),
        pltpu.SemaphoreType.DMA,
    ],
)
def _segsum_kernel(zflat_hbm, src_hbm, dst_hbm, out_hbm,
                   s_sh, src_v, dst_v, rows_v, sem):
    c = lax.axis_index("c")
    t = lax.axis_index("s")
    for p in range(2):
        part = c * 2 + p
        zoff = part * NPAD
        pltpu.sync_copy(zflat_hbm.at[pl.ds(zoff + t * TPR, TPR)],
                        s_sh.at[pl.ds(t * TPR, TPR)])
        plsc.subcore_barrier()

        def body(g, _):
            row0 = t * RPT + g * GB
            pltpu.sync_copy(src_hbm.at[pl.ds(row0, GB)], src_v)
            pltpu.sync_copy(dst_hbm.at[pl.ds(row0, GB)], dst_v)
            for j in range(GB):
                for k in range(8):
                    sl = (j, pl.ds(k * 16, 16))
                    src_v[sl] = src_v[sl] + zoff
            handles = [
                pltpu.async_copy(zflat_hbm.at[src_v.at[j]], rows_v.at[j], sem)
                for j in range(GB)
            ]
            for j in range(GB):
                handles[j].wait()
                pltpu.sync_copy(rows_v.at[j], s_sh.at[dst_v.at[j]], add=True)
            return _

        lax.fori_loop(0, NGRP, body, None)
        plsc.subcore_barrier()
        pltpu.sync_copy(s_sh.at[pl.ds(t * TPR, TPR)],
                        out_hbm.at[pl.ds(zoff + t * TPR, TPR)])
        plsc.subcore_barrier()


def _unpack_lanes(blk, w):
    return jnp.concatenate(
        [blk[:, u * w:(u + 1) * w] for u in range(8)], axis=0)


def _scale_body(x_ref, dpk_ref, out_ref):
    dinv = _unpack_lanes(dpk_ref[...], 16)[:, 0:1]
    z = x_ref[...] * dinv
    for p in range(NPARTS):
        zp = z[:, p * 16:(p + 1) * 16]
        out_ref[p] = jnp.concatenate(
            [zp[u * (TPR // 8):(u + 1) * (TPR // 8), :] for u in range(8)],
            axis=1)


def _scale_split(x_pad, dinvpk):
    return pl.pallas_call(
        _scale_body,
        grid=(NPAD // TPR,),
        in_specs=[
            pl.BlockSpec((TPR, 64), lambda i: (i, 0)),
            pl.BlockSpec((TPR // 8, 128), lambda i: (i, 0)),
        ],
        out_specs=pl.BlockSpec((NPARTS, TPR // 8, 128), lambda i: (0, i, 0)),
        out_shape=jax.ShapeDtypeStruct((NPARTS, NPAD // 8, 128), jnp.float32),
    )(x_pad, dinvpk)


def _head_body(st_ref, dpk_ref, wc_ref, bc_ref, wl_ref, bl_ref,
               out_ref, acc_ref):
    i = pl.program_id(0)

    g = jnp.dot(_unpack_lanes(st_ref[0], 16), wc_ref[0],
                preferred_element_type=jnp.float32)
    for p in range(1, NPARTS):
        g += jnp.dot(_unpack_lanes(st_ref[p], 16), wc_ref[p],
                     preferred_element_type=jnp.float32)
    dinv = _unpack_lanes(dpk_ref[...], 16)[:, 0:1]
    h = jnp.maximum(g * dinv + bc_ref[...], 0.0)
    rows = lax.broadcasted_iota(jnp.int32, (TPR, 1), 0) + i * TPR
    h = jnp.where(rows < N, h, 0.0)
    partial = jnp.sum(h.reshape(TPR // 8, 8, 128), axis=0)

    @pl.when(i == 0)
    def _():
        acc_ref[...] = partial

    @pl.when(i > 0)
    def _():
        acc_ref[...] += partial

    @pl.when(i == (NPAD // TPR) - 1)
    def _():
        emb = jnp.sum(acc_ref[...], axis=0, keepdims=True) * (1.0 / N)
        out_ref[...] = jnp.tanh(
            jnp.dot(emb, wl_ref[...], preferred_element_type=jnp.float32)
            + bl_ref[...])


def _head(stpk, dinvpk, wc4, bc2d, wl, bl2d):
    return pl.pallas_call(
        _head_body,
        grid=(NPAD // TPR,),
        in_specs=[
            pl.BlockSpec((NPARTS, TPR // 8, 128), lambda i: (0, i, 0)),
            pl.BlockSpec((TPR // 8, 128), lambda i: (i, 0)),
            pl.BlockSpec((NPARTS, 16, 128), lambda i: (0, 0, 0)),
            pl.BlockSpec((1, 128), lambda i: (0, 0)),
            pl.BlockSpec((128, 128), lambda i: (0, 0)),
            pl.BlockSpec((1, 128), lambda i: (0, 0)),
        ],
        out_specs=pl.BlockSpec((1, 128), lambda i: (0, 0)),
        out_shape=jax.ShapeDtypeStruct((1, 128), jnp.float32),
        scratch_shapes=[pltpu.VMEM((8, 128), jnp.float32)],
    )(stpk, dinvpk, wc4, bc2d, wl, bl2d)


def kernel(edge_index, W_conv, b_conv, W_lin, b_lin):
    in_feat = W_conv.shape[0]
    src = edge_index[0].astype(jnp.int32)
    dst = edge_index[1].astype(jnp.int32)
    npad_e = EPAD - E
    pad_src = N + (jnp.arange(npad_e, dtype=jnp.int32) % 8)
    pad_dst = (N + 8) + (jnp.arange(npad_e, dtype=jnp.int32) % (NPAD - N - 8))
    def perm(n):
        blk, r = n // TPR, n % TPR
        return blk * TPR + 8 * (r % (TPR // 8)) + r // (TPR // 8)

    dst_plain = jnp.concatenate([dst, pad_dst])
    srcp = perm(jnp.concatenate([src, pad_src]))
    src2d = srcp.reshape(EROWS, 128)
    dst2d = perm(dst_plain).reshape(EROWS, 128)

    degc = _deg_kernel(dst_plain.reshape(EROWS, 128))
    deg = degc[0] + degc[1] - 1.0
    dinv1d = lax.rsqrt(deg)
    dinvpk = jnp.repeat(
        jnp.swapaxes(dinv1d.reshape(16, 8, TPR // 8), 1, 2).reshape(-1), 16
    ).reshape(NPAD // 8, 128)

    x = jax.random.normal(jax.random.key(42), (N, in_feat), dtype=jnp.float32)
    x_pad = jnp.zeros((NPAD, 64), jnp.float32).at[:N, :in_feat].set(x)

    zpk = _scale_split(x_pad, dinvpk)
    st = _segsum_kernel(zpk.reshape(ZROWS, 16), src2d, dst2d)
    stpk = st.reshape(NPARTS, NPAD // 8, 128)

    wc4 = jnp.zeros((64, 128), jnp.float32).at[:in_feat].set(W_conv)
    wc4 = wc4.reshape(NPARTS, 16, 128)
    out = _head(stpk, dinvpk, wc4, b_conv[None, :], W_lin, b_lin[None, :])
    return out
